# Initial kernel scaffold; baseline (speedup 1.0000x reference)
import jax
import jax.numpy as jnp
from jax.experimental import pallas as pl


def kernel(x):
    raise NotImplementedError("write your pallas kernel here")


# speedup vs baseline: 2.5801x; 2.5801x over previous
import functools

import numpy as np
import jax
import jax.numpy as jnp
from jax import lax
from jax.experimental import pallas as pl
from jax.experimental.pallas import tpu as pltpu

L = 8192
C = 64
CC = 32
NB = 32
V, U = 128, 64
M, W = 128, 16
TOP_K = 5
CUT = 1637

_v = np.arange(V)
_u = np.arange(U)
_m = np.arange(M)
_w = np.arange(W)

_S1 = np.exp(-2j * np.pi * np.outer(_m, _v) / M)
_TW = np.exp(-2j * np.pi * np.outer(_m, _u) / L)
_S2 = np.exp(-2j * np.pi * np.outer(_u, _w) / U)

S1RE = np.ascontiguousarray(_S1.real, np.float32)
S1IM = np.ascontiguousarray(_S1.imag, np.float32)
TWRE = np.ascontiguousarray(_TW.real, np.float32)
TWIM = np.ascontiguousarray(_TW.imag, np.float32)
S2RE = np.ascontiguousarray(_S2.real, np.float32)
S2IM = np.ascontiguousarray(_S2.imag, np.float32)

_kval = (128 * _w[None, :] + _m[:, None]).astype(np.int32)
KVAL = np.ascontiguousarray(_kval)
KVALID = np.ascontiguousarray((_kval >= 1) & (_kval <= CUT))


def _dft_body(x_ref, s1re, s1im, twre, twim, s2re, s2im, kval_ref, kvalid_ref,
              season_ref, trend_ref):
    xb = x_ref[0]
    xT = jnp.transpose(xb)

    hp = jax.lax.Precision.HIGHEST
    twr = twre[...].reshape(M, 1, U)
    twi = twim[...].reshape(M, 1, U)
    g_re = s2re[...]
    g_im = s2im[...]
    cre_h, cim_h = [], []
    for h in range(C // CC):
        xT_h = lax.slice_in_dim(xT, h * CC, (h + 1) * CC, axis=0)
        xr = jnp.transpose(xT_h.reshape(CC, V, U), (1, 0, 2)).reshape(V, CC * U)
        are = lax.dot(s1re[...], xr, precision=hp).reshape(M, CC, U)
        aim = lax.dot(s1im[...], xr, precision=hp).reshape(M, CC, U)
        bre = are * twr - aim * twi
        bim = are * twi + aim * twr
        b2re = bre.reshape(M * CC, U)
        b2im = bim.reshape(M * CC, U)
        cre_h.append((lax.dot(b2re, g_re, precision=hp)
                      - lax.dot(b2im, g_im, precision=hp)).reshape(M, CC, W))
        cim_h.append((lax.dot(b2re, g_im, precision=hp)
                      + lax.dot(b2im, g_re, precision=hp)).reshape(M, CC, W))
    cre = jnp.concatenate(cre_h, axis=1)
    cim = jnp.concatenate(cim_h, axis=1)

    mag2 = cre * cre + cim * cim
    kval = kval_ref[...].reshape(M, 1, W)
    kvalid = kvalid_ref[...].reshape(M, 1, W)
    mag2 = jnp.where(kvalid, mag2, -1.0)

    a_ar = lax.broadcasted_iota(jnp.int32, (C, U), 1)
    v_ar = lax.broadcasted_iota(jnp.int32, (C, V), 1)
    season = jnp.zeros((C, U, V), jnp.float32)
    two_over_l = jnp.float32(2.0 / L)
    for _ in range(TOP_K):
        mx = jnp.max(jnp.max(mag2, axis=2), axis=0)
        sel = mag2 == mx[None, :, None]
        kcand = jnp.where(sel, kval, jnp.int32(4096))
        ksel = jnp.min(jnp.min(kcand, axis=2), axis=0)
        hit = kval == ksel[None, :, None]
        re = jnp.sum(jnp.sum(jnp.where(hit, cre, 0.0), axis=2), axis=0)
        im = jnp.sum(jnp.sum(jnp.where(hit, cim, 0.0), axis=2), axis=0)
        mag2 = jnp.where(hit, -1.0, mag2)
        re = re * two_over_l
        im = im * two_over_l
        ka = (ksel[:, None] * a_ar) & 63
        kv = (ksel[:, None] * v_ar) & 8191
        aa = ka.astype(jnp.float32) * jnp.float32(2.0 * np.pi / 64.0)
        bb = kv.astype(jnp.float32) * jnp.float32(2.0 * np.pi / 8192.0)
        ca, sa = jnp.cos(aa), jnp.sin(aa)
        cb, sb = jnp.cos(bb), jnp.sin(bb)
        p1 = (re[:, None] * ca - im[:, None] * sa)[:, :, None]
        p2 = -(re[:, None] * sa + im[:, None] * ca)[:, :, None]
        season = season + p1 * cb[:, None, :] + p2 * sb[:, None, :]

    season = season.reshape(C, L)
    season_ref[0] = season
    trend_ref[0] = xT - season


@jax.jit
def kernel(x):
    grid = (NB,)
    out = pl.pallas_call(
        _dft_body,
        grid=grid,
        in_specs=[
            pl.BlockSpec((1, L, C), lambda b: (b, 0, 0)),
            pl.BlockSpec((M, V), lambda b: (0, 0)),
            pl.BlockSpec((M, V), lambda b: (0, 0)),
            pl.BlockSpec((M, U), lambda b: (0, 0)),
            pl.BlockSpec((M, U), lambda b: (0, 0)),
            pl.BlockSpec((U, W), lambda b: (0, 0)),
            pl.BlockSpec((U, W), lambda b: (0, 0)),
            pl.BlockSpec((M, W), lambda b: (0, 0)),
            pl.BlockSpec((M, W), lambda b: (0, 0)),
        ],
        out_specs=[
            pl.BlockSpec((1, C, L), lambda b: (b, 0, 0)),
            pl.BlockSpec((1, C, L), lambda b: (b, 0, 0)),
        ],
        out_shape=[
            jax.ShapeDtypeStruct((NB, C, L), jnp.float32),
            jax.ShapeDtypeStruct((NB, C, L), jnp.float32),
        ],
        compiler_params=pltpu.CompilerParams(
            dimension_semantics=("parallel",),
        ),
    )(x, S1RE, S1IM, TWRE, TWIM, S2RE, S2IM, KVAL, KVALID)
    return (out[0], out[1])


# speedup vs baseline: 3.9721x; 1.5395x over previous
import functools

import numpy as np
import jax
import jax.numpy as jnp
from jax import lax
from jax.experimental import pallas as pl
from jax.experimental.pallas import tpu as pltpu
from jax.experimental.pallas import tpu_sc as plsc

L = 8192
C = 64
CC = 32
NB = 32
V, U = 128, 64
M, W = 128, 16
TOP_K = 5
CUT = 1637
NSER = NB * C

_v = np.arange(V)
_u = np.arange(U)
_m = np.arange(M)
_w = np.arange(W)

_S1 = np.exp(-2j * np.pi * np.outer(_m, _v) / M)
_TW = np.exp(-2j * np.pi * np.outer(_m, _u) / L)
_S2 = np.exp(-2j * np.pi * np.outer(_u, _w) / U)

S1RE = np.ascontiguousarray(_S1.real, np.float32)
S1IM = np.ascontiguousarray(_S1.imag, np.float32)
TWRE = np.ascontiguousarray(_TW.real, np.float32)
TWIM = np.ascontiguousarray(_TW.imag, np.float32)
S2RE = np.ascontiguousarray(_S2.real, np.float32)
S2IM = np.ascontiguousarray(_S2.imag, np.float32)

_kval = (128 * _w[None, :] + _m[:, None]).astype(np.int32)
KVALID = np.ascontiguousarray((_kval >= 1) & (_kval <= CUT))


def _dft_body(x_ref, s1re, s1im, twre, twim, s2re, s2im, kvalid_ref, out_ref):
    xb = x_ref[0]
    xT = jnp.transpose(xb)

    hp = jax.lax.Precision.HIGHEST
    twr = twre[...].reshape(M, 1, U)
    twi = twim[...].reshape(M, 1, U)
    g_re = s2re[...]
    g_im = s2im[...]
    cre_h, cim_h = [], []
    for h in range(C // CC):
        xT_h = lax.slice_in_dim(xT, h * CC, (h + 1) * CC, axis=0)
        xr = jnp.transpose(xT_h.reshape(CC, V, U), (1, 0, 2)).reshape(V, CC * U)
        are = lax.dot(s1re[...], xr, precision=hp).reshape(M, CC, U)
        aim = lax.dot(s1im[...], xr, precision=hp).reshape(M, CC, U)
        bre = are * twr - aim * twi
        bim = are * twi + aim * twr
        b2re = bre.reshape(M * CC, U)
        b2im = bim.reshape(M * CC, U)
        cre_h.append((lax.dot(b2re, g_re, precision=hp)
                      - lax.dot(b2im, g_im, precision=hp)).reshape(M, CC, W))
        cim_h.append((lax.dot(b2re, g_im, precision=hp)
                      + lax.dot(b2im, g_re, precision=hp)).reshape(M, CC, W))
    cre = jnp.concatenate(cre_h, axis=1)
    cim = jnp.concatenate(cim_h, axis=1)

    mag2 = cre * cre + cim * cim
    kvalid = kvalid_ref[...].reshape(M, 1, W)
    mag2 = jnp.where(kvalid, mag2, -1.0)
    out_ref[0] = jnp.concatenate([mag2, cre, cim], axis=2)


def _tc_spectrum(x):
    return pl.pallas_call(
        _dft_body,
        grid=(NB,),
        in_specs=[
            pl.BlockSpec((1, L, C), lambda b: (b, 0, 0)),
            pl.BlockSpec((M, V), lambda b: (0, 0)),
            pl.BlockSpec((M, V), lambda b: (0, 0)),
            pl.BlockSpec((M, U), lambda b: (0, 0)),
            pl.BlockSpec((M, U), lambda b: (0, 0)),
            pl.BlockSpec((U, W), lambda b: (0, 0)),
            pl.BlockSpec((U, W), lambda b: (0, 0)),
            pl.BlockSpec((M, W), lambda b: (0, 0)),
        ],
        out_specs=pl.BlockSpec((1, M, C, 3 * W), lambda b: (b, 0, 0, 0)),
        out_shape=jax.ShapeDtypeStruct((NB, M, C, 3 * W), jnp.float32),
        compiler_params=pltpu.CompilerParams(
            dimension_semantics=("parallel",),
        ),
    )(x, S1RE, S1IM, TWRE, TWIM, S2RE, S2IM, KVALID)


def _take16(x, idx):
    return x.at[idx].get(mode="promise_in_bounds")


def _sc_body(fused_hbm, out_hbm, buf, row, iscr):
    nc = 2
    wid = lax.axis_index("s") * nc + lax.axis_index("c")
    liota = lax.iota(jnp.int32, 16)

    def xlane_max(x):
        for sh in (8, 4, 2, 1):
            x = jnp.maximum(x, _take16(x, liota ^ sh))
        return x

    def xlane_min(x):
        for sh in (8, 4, 2, 1):
            x = jnp.minimum(x, _take16(x, liota ^ sh))
        return x

    def series_body(i, carry):
        pltpu.sync_copy(fused_hbm.at[wid, :, i, :], buf)

        kvec = jnp.zeros((16,), jnp.float32)
        revec = jnp.zeros((16,), jnp.float32)
        imvec = jnp.zeros((16,), jnp.float32)
        for j in range(TOP_K):
            def scan_body(m, mxam):
                mx, am = mxam
                vals = buf[m, 0:16]
                upd = vals > mx
                mx = jnp.where(upd, vals, mx)
                am = jnp.where(upd, jnp.full((16,), m, jnp.int32), am)
                return mx, am

            mx0 = jnp.full((16,), -2.0, jnp.float32)
            am0 = jnp.zeros((16,), jnp.int32)
            mx, am = lax.fori_loop(0, M, scan_body, (mx0, am0))
            gmax_v = xlane_max(mx)
            kcand = jnp.where(mx == gmax_v, 128 * liota + am, jnp.int32(4096))
            ks_v = xlane_min(kcand)
            ks = ks_v[0]
            m_star = ks & 127
            w_star_v = ks_v >> 7
            lane_hit = liota == w_star_v
            magrow = buf[m_star, 0:16]
            rerow = buf[m_star, 16:32]
            imrow = buf[m_star, 32:48]
            re_v = _take16(rerow, w_star_v)
            im_v = _take16(imrow, w_star_v)
            buf[m_star, 0:16] = jnp.where(lane_hit, -1.0, magrow)
            jsel = liota == j
            kvec = jnp.where(jsel, ks_v.astype(jnp.float32), kvec)
            revec = jnp.where(jsel, re_v, revec)
            imvec = jnp.where(jsel, im_v, imvec)
        row[0] = kvec
        row[1] = revec
        row[2] = imvec
        pltpu.sync_copy(row, out_hbm.at[wid * C + i])
        return carry

    lax.fori_loop(0, C, series_body, 0)


@functools.partial(
    pl.kernel,
    mesh=plsc.VectorSubcoreMesh(core_axis_name="c", subcore_axis_name="s"),
    out_type=jax.ShapeDtypeStruct((NSER, 3, 16), jnp.float32),
    scratch_types=[
        pltpu.VMEM((M, 3 * W), jnp.float32),
        pltpu.VMEM((3, 16), jnp.float32),
        pltpu.VMEM((1, 16), jnp.int32),
    ],
)
def _sc_select(fused_hbm, out_hbm, buf, row, iscr):
    _sc_body(fused_hbm, out_hbm, buf, row, iscr)


def _recon_body(x_ref, sel_ref, season_ref, trend_ref):
    xb = x_ref[0]
    xT = jnp.transpose(xb)
    kf = sel_ref[:, 0, :]
    rf = sel_ref[:, 1, :]
    imf = sel_ref[:, 2, :]

    a_ar = lax.broadcasted_iota(jnp.int32, (C, U), 1)
    v_ar = lax.broadcasted_iota(jnp.int32, (C, V), 1)
    j_ar = lax.broadcasted_iota(jnp.int32, (C, 16), 1)
    season = jnp.zeros((C, U, V), jnp.float32)
    two_over_l = jnp.float32(2.0 / L)
    for j in range(TOP_K):
        jhit = j_ar == j
        ksel = jnp.sum(jnp.where(jhit, kf, 0.0), axis=1).astype(jnp.int32)
        re = jnp.sum(jnp.where(jhit, rf, 0.0), axis=1) * two_over_l
        im = jnp.sum(jnp.where(jhit, imf, 0.0), axis=1) * two_over_l
        ka = (ksel[:, None] * a_ar) & 63
        kv = (ksel[:, None] * v_ar) & 8191
        aa = ka.astype(jnp.float32) * jnp.float32(2.0 * np.pi / 64.0)
        bb = kv.astype(jnp.float32) * jnp.float32(2.0 * np.pi / 8192.0)
        ca, sa = jnp.cos(aa), jnp.sin(aa)
        cb, sb = jnp.cos(bb), jnp.sin(bb)
        p1 = (re[:, None] * ca - im[:, None] * sa)[:, :, None]
        p2 = -(re[:, None] * sa + im[:, None] * ca)[:, :, None]
        season = season + p1 * cb[:, None, :] + p2 * sb[:, None, :]

    season = season.reshape(C, L)
    season_ref[0] = season
    trend_ref[0] = xT - season


def _tc_recon(x, sel):
    return pl.pallas_call(
        _recon_body,
        grid=(NB,),
        in_specs=[
            pl.BlockSpec((1, L, C), lambda b: (b, 0, 0)),
            pl.BlockSpec((C, 3, 16), lambda b: (b, 0, 0)),
        ],
        out_specs=[
            pl.BlockSpec((1, C, L), lambda b: (b, 0, 0)),
            pl.BlockSpec((1, C, L), lambda b: (b, 0, 0)),
        ],
        out_shape=[
            jax.ShapeDtypeStruct((NB, C, L), jnp.float32),
            jax.ShapeDtypeStruct((NB, C, L), jnp.float32),
        ],
        compiler_params=pltpu.CompilerParams(
            dimension_semantics=("parallel",),
        ),
    )(x, sel)


@jax.jit
def kernel(x):
    fused = _tc_spectrum(x)
    sel = _sc_select(fused)
    season, trend = _tc_recon(x, sel)
    return (season, trend)


# speedup vs baseline: 4.0120x; 1.0101x over previous
import functools

import numpy as np
import jax
import jax.numpy as jnp
from jax import lax
from jax.experimental import pallas as pl
from jax.experimental.pallas import tpu as pltpu
from jax.experimental.pallas import tpu_sc as plsc

L = 8192
C = 64
CC = 32
NB = 32
V, U = 128, 64
M, W = 128, 16
TOP_K = 5
CUT = 1637
NSER = NB * C

_v = np.arange(V)
_u = np.arange(U)
_m = np.arange(M)
_w = np.arange(W)

_S1 = np.exp(-2j * np.pi * np.outer(_m, _v) / M)
_TW = np.exp(-2j * np.pi * np.outer(_m, _u) / L)
_S2 = np.exp(-2j * np.pi * np.outer(_u, _w) / U)

S1RE = np.ascontiguousarray(_S1.real, np.float32)
S1IM = np.ascontiguousarray(_S1.imag, np.float32)
TWRE = np.ascontiguousarray(_TW.real, np.float32)
TWIM = np.ascontiguousarray(_TW.imag, np.float32)
S2RE = np.ascontiguousarray(_S2.real, np.float32)
S2IM = np.ascontiguousarray(_S2.imag, np.float32)

_kval = (128 * _w[None, :] + _m[:, None]).astype(np.int32)
KVALID = np.ascontiguousarray((_kval >= 1) & (_kval <= CUT))


def _dft_body(x_ref, s1re, s1im, twre, twim, s2re, s2im, kvalid_ref, out_ref):
    xb = x_ref[0]
    xrp = jnp.transpose(xb.reshape(V, U, C), (0, 2, 1)).reshape(V, C * U)

    hp = jax.lax.Precision.HIGHEST
    are = lax.dot(s1re[...], xrp, precision=hp).reshape(M, C, U)
    aim = lax.dot(s1im[...], xrp, precision=hp).reshape(M, C, U)
    twr = twre[...].reshape(M, 1, U)
    twi = twim[...].reshape(M, 1, U)
    bre = are * twr - aim * twi
    bim = are * twi + aim * twr
    b2re = bre.reshape(M * C, U)
    b2im = bim.reshape(M * C, U)
    g_re = s2re[...]
    g_im = s2im[...]
    cre = (lax.dot(b2re, g_re, precision=hp)
           - lax.dot(b2im, g_im, precision=hp)).reshape(M, C, W)
    cim = (lax.dot(b2re, g_im, precision=hp)
           + lax.dot(b2im, g_re, precision=hp)).reshape(M, C, W)

    mag2 = cre * cre + cim * cim
    kvalid = kvalid_ref[...].reshape(M, 1, W)
    mag2 = jnp.where(kvalid, mag2, -1.0)
    out_ref[0] = jnp.concatenate([mag2, cre, cim], axis=2)


def _tc_spectrum(x):
    return pl.pallas_call(
        _dft_body,
        grid=(NB,),
        in_specs=[
            pl.BlockSpec((1, L, C), lambda b: (b, 0, 0)),
            pl.BlockSpec((M, V), lambda b: (0, 0)),
            pl.BlockSpec((M, V), lambda b: (0, 0)),
            pl.BlockSpec((M, U), lambda b: (0, 0)),
            pl.BlockSpec((M, U), lambda b: (0, 0)),
            pl.BlockSpec((U, W), lambda b: (0, 0)),
            pl.BlockSpec((U, W), lambda b: (0, 0)),
            pl.BlockSpec((M, W), lambda b: (0, 0)),
        ],
        out_specs=pl.BlockSpec((1, M, C, 3 * W), lambda b: (b, 0, 0, 0)),
        out_shape=jax.ShapeDtypeStruct((NB, M, C, 3 * W), jnp.float32),
        compiler_params=pltpu.CompilerParams(
            dimension_semantics=("parallel",),
            vmem_limit_bytes=62 * 1024 * 1024,
        ),
    )(x, S1RE, S1IM, TWRE, TWIM, S2RE, S2IM, KVALID)


def _take16(x, idx):
    return x.at[idx].get(mode="promise_in_bounds")


def _sc_body(fused_hbm, out_hbm, buf, row, iscr):
    nc = 2
    wid = lax.axis_index("s") * nc + lax.axis_index("c")
    liota = lax.iota(jnp.int32, 16)

    def xlane_max(x):
        for sh in (8, 4, 2, 1):
            x = jnp.maximum(x, _take16(x, liota ^ sh))
        return x

    def xlane_min(x):
        for sh in (8, 4, 2, 1):
            x = jnp.minimum(x, _take16(x, liota ^ sh))
        return x

    def series_body(i, carry):
        pltpu.sync_copy(fused_hbm.at[wid, :, i, :], buf)

        kvec = jnp.zeros((16,), jnp.float32)
        revec = jnp.zeros((16,), jnp.float32)
        imvec = jnp.zeros((16,), jnp.float32)
        for j in range(TOP_K):
            def scan_body(m, mxam):
                mx, am = mxam
                vals = buf[m, 0:16]
                upd = vals > mx
                mx = jnp.where(upd, vals, mx)
                am = jnp.where(upd, jnp.full((16,), m, jnp.int32), am)
                return mx, am

            mx0 = jnp.full((16,), -2.0, jnp.float32)
            am0 = jnp.zeros((16,), jnp.int32)
            mx, am = lax.fori_loop(0, M, scan_body, (mx0, am0))
            gmax_v = xlane_max(mx)
            kcand = jnp.where(mx == gmax_v, 128 * liota + am, jnp.int32(4096))
            ks_v = xlane_min(kcand)
            ks = ks_v[0]
            m_star = ks & 127
            w_star_v = ks_v >> 7
            lane_hit = liota == w_star_v
            magrow = buf[m_star, 0:16]
            rerow = buf[m_star, 16:32]
            imrow = buf[m_star, 32:48]
            re_v = _take16(rerow, w_star_v)
            im_v = _take16(imrow, w_star_v)
            buf[m_star, 0:16] = jnp.where(lane_hit, -1.0, magrow)
            jsel = liota == j
            kvec = jnp.where(jsel, ks_v.astype(jnp.float32), kvec)
            revec = jnp.where(jsel, re_v, revec)
            imvec = jnp.where(jsel, im_v, imvec)
        row[0] = kvec
        row[1] = revec
        row[2] = imvec
        pltpu.sync_copy(row, out_hbm.at[wid * C + i])
        return carry

    lax.fori_loop(0, C, series_body, 0)


@functools.partial(
    pl.kernel,
    mesh=plsc.VectorSubcoreMesh(core_axis_name="c", subcore_axis_name="s"),
    out_type=jax.ShapeDtypeStruct((NSER, 3, 16), jnp.float32),
    scratch_types=[
        pltpu.VMEM((M, 3 * W), jnp.float32),
        pltpu.VMEM((3, 16), jnp.float32),
        pltpu.VMEM((1, 16), jnp.int32),
    ],
)
def _sc_select(fused_hbm, out_hbm, buf, row, iscr):
    _sc_body(fused_hbm, out_hbm, buf, row, iscr)


def _recon_body(x_ref, sel_ref, season_ref, trend_ref):
    xb = x_ref[0]
    xT = jnp.transpose(xb)
    kf = sel_ref[:, 0, :]
    rf = sel_ref[:, 1, :]
    imf = sel_ref[:, 2, :]

    a_ar = lax.broadcasted_iota(jnp.int32, (C, U), 1)
    v_ar = lax.broadcasted_iota(jnp.int32, (C, V), 1)
    j_ar = lax.broadcasted_iota(jnp.int32, (C, 16), 1)
    season = jnp.zeros((C, U, V), jnp.float32)
    two_over_l = jnp.float32(2.0 / L)
    for j in range(TOP_K):
        jhit = j_ar == j
        ksel = jnp.sum(jnp.where(jhit, kf, 0.0), axis=1).astype(jnp.int32)
        re = jnp.sum(jnp.where(jhit, rf, 0.0), axis=1) * two_over_l
        im = jnp.sum(jnp.where(jhit, imf, 0.0), axis=1) * two_over_l
        ka = (ksel[:, None] * a_ar) & 63
        kv = (ksel[:, None] * v_ar) & 8191
        aa = ka.astype(jnp.float32) * jnp.float32(2.0 * np.pi / 64.0)
        bb = kv.astype(jnp.float32) * jnp.float32(2.0 * np.pi / 8192.0)
        ca, sa = jnp.cos(aa), jnp.sin(aa)
        cb, sb = jnp.cos(bb), jnp.sin(bb)
        p1 = (re[:, None] * ca - im[:, None] * sa)[:, :, None]
        p2 = -(re[:, None] * sa + im[:, None] * ca)[:, :, None]
        season = season + p1 * cb[:, None, :] + p2 * sb[:, None, :]

    season = season.reshape(C, L)
    season_ref[0] = season
    trend_ref[0] = xT - season


def _tc_recon(x, sel):
    return pl.pallas_call(
        _recon_body,
        grid=(NB,),
        in_specs=[
            pl.BlockSpec((1, L, C), lambda b: (b, 0, 0)),
            pl.BlockSpec((C, 3, 16), lambda b: (b, 0, 0)),
        ],
        out_specs=[
            pl.BlockSpec((1, C, L), lambda b: (b, 0, 0)),
            pl.BlockSpec((1, C, L), lambda b: (b, 0, 0)),
        ],
        out_shape=[
            jax.ShapeDtypeStruct((NB, C, L), jnp.float32),
            jax.ShapeDtypeStruct((NB, C, L), jnp.float32),
        ],
        compiler_params=pltpu.CompilerParams(
            dimension_semantics=("parallel",),
        ),
    )(x, sel)


@jax.jit
def kernel(x):
    fused = _tc_spectrum(x)
    sel = _sc_select(fused)
    season, trend = _tc_recon(x, sel)
    return (season, trend)


# speedup vs baseline: 4.3508x; 1.0844x over previous
import functools

import numpy as np
import jax
import jax.numpy as jnp
from jax import lax
from jax.experimental import pallas as pl
from jax.experimental.pallas import tpu as pltpu
from jax.experimental.pallas import tpu_sc as plsc

L = 8192
C = 64
CC = 32
NB = 32
V, U = 128, 64
M, W = 128, 16
TOP_K = 5
CUT = 1637
NSER = NB * C

_v = np.arange(V)
_u = np.arange(U)
_m = np.arange(M)
_w = np.arange(W)

_S1 = np.exp(-2j * np.pi * np.outer(_m, _v) / M)
_TW = np.exp(-2j * np.pi * np.outer(_m, _u) / L)
_S2 = np.exp(-2j * np.pi * np.outer(_u, _w) / U)

S1RE = np.ascontiguousarray(_S1.real, np.float32)
S1IM = np.ascontiguousarray(_S1.imag, np.float32)
TWRE = np.ascontiguousarray(_TW.real, np.float32)
TWIM = np.ascontiguousarray(_TW.imag, np.float32)
S2RE = np.ascontiguousarray(_S2.real, np.float32)
S2IM = np.ascontiguousarray(_S2.imag, np.float32)

_kval = (128 * _w[None, :] + _m[:, None]).astype(np.int32)
KVALID = np.ascontiguousarray((_kval >= 1) & (_kval <= CUT))


def _dft_body(x_ref, s1re, s1im, twre, twim, s2re, s2im, kvalid_ref, out_ref):
    xb = x_ref[0]
    xrp = jnp.transpose(xb.reshape(V, U, C), (0, 2, 1)).reshape(V, C * U)

    hp = jax.lax.Precision.HIGHEST
    are = lax.dot(s1re[...], xrp, precision=hp).reshape(M, C, U)
    aim = lax.dot(s1im[...], xrp, precision=hp).reshape(M, C, U)
    twr = twre[...].reshape(M, 1, U)
    twi = twim[...].reshape(M, 1, U)
    bre = are * twr - aim * twi
    bim = are * twi + aim * twr
    b2re = bre.reshape(M * C, U)
    b2im = bim.reshape(M * C, U)
    g2 = jnp.concatenate([s2re[...], s2im[...]], axis=1)
    p1 = lax.dot(b2re, g2, precision=hp)
    p2 = lax.dot(b2im, g2, precision=hp)
    cre = (p1[:, 0:W] - p2[:, W:2 * W]).reshape(M, C, W)
    cim = (p1[:, W:2 * W] + p2[:, 0:W]).reshape(M, C, W)

    mag2 = cre * cre + cim * cim
    kvalid = kvalid_ref[...].reshape(M, 1, W)
    mag2 = jnp.where(kvalid, mag2, -1.0)
    out_ref[0] = jnp.concatenate([mag2, cre, cim], axis=2)


def _tc_spectrum(x):
    return pl.pallas_call(
        _dft_body,
        grid=(NB,),
        in_specs=[
            pl.BlockSpec((1, L, C), lambda b: (b, 0, 0)),
            pl.BlockSpec((M, V), lambda b: (0, 0)),
            pl.BlockSpec((M, V), lambda b: (0, 0)),
            pl.BlockSpec((M, U), lambda b: (0, 0)),
            pl.BlockSpec((M, U), lambda b: (0, 0)),
            pl.BlockSpec((U, W), lambda b: (0, 0)),
            pl.BlockSpec((U, W), lambda b: (0, 0)),
            pl.BlockSpec((M, W), lambda b: (0, 0)),
        ],
        out_specs=pl.BlockSpec((1, M, C, 3 * W), lambda b: (b, 0, 0, 0)),
        out_shape=jax.ShapeDtypeStruct((NB, M, C, 3 * W), jnp.float32),
        compiler_params=pltpu.CompilerParams(
            dimension_semantics=("parallel",),
            vmem_limit_bytes=62 * 1024 * 1024,
        ),
    )(x, S1RE, S1IM, TWRE, TWIM, S2RE, S2IM, KVALID)


def _take16(x, idx):
    return x.at[idx].get(mode="promise_in_bounds")


def _sc_body(fused_hbm, out_hbm, buf, row, iscr):
    nc = 2
    wid = lax.axis_index("s") * nc + lax.axis_index("c")
    liota = lax.iota(jnp.int32, 16)

    def xlane_max(x):
        for sh in (8, 4, 2, 1):
            x = jnp.maximum(x, _take16(x, liota ^ sh))
        return x

    def xlane_min(x):
        for sh in (8, 4, 2, 1):
            x = jnp.minimum(x, _take16(x, liota ^ sh))
        return x

    def series_body(i, carry):
        pltpu.sync_copy(fused_hbm.at[wid, :, i, :], buf)

        kvec = jnp.zeros((16,), jnp.float32)
        revec = jnp.zeros((16,), jnp.float32)
        imvec = jnp.zeros((16,), jnp.float32)
        for j in range(TOP_K):
            def scan_body(m, mxam):
                mx, am = mxam
                vals = buf[m, 0:16]
                upd = vals > mx
                mx = jnp.where(upd, vals, mx)
                am = jnp.where(upd, jnp.full((16,), m, jnp.int32), am)
                return mx, am

            mx0 = jnp.full((16,), -2.0, jnp.float32)
            am0 = jnp.zeros((16,), jnp.int32)
            mx, am = lax.fori_loop(0, M, scan_body, (mx0, am0))
            gmax_v = xlane_max(mx)
            kcand = jnp.where(mx == gmax_v, 128 * liota + am, jnp.int32(4096))
            ks_v = xlane_min(kcand)
            ks = ks_v[0]
            m_star = ks & 127
            w_star_v = ks_v >> 7
            lane_hit = liota == w_star_v
            magrow = buf[m_star, 0:16]
            rerow = buf[m_star, 16:32]
            imrow = buf[m_star, 32:48]
            re_v = _take16(rerow, w_star_v)
            im_v = _take16(imrow, w_star_v)
            buf[m_star, 0:16] = jnp.where(lane_hit, -1.0, magrow)
            jsel = liota == j
            kvec = jnp.where(jsel, ks_v.astype(jnp.float32), kvec)
            revec = jnp.where(jsel, re_v, revec)
            imvec = jnp.where(jsel, im_v, imvec)
        row[0] = kvec
        row[1] = revec
        row[2] = imvec
        pltpu.sync_copy(row, out_hbm.at[wid * C + i])
        return carry

    lax.fori_loop(0, C, series_body, 0)


@functools.partial(
    pl.kernel,
    mesh=plsc.VectorSubcoreMesh(core_axis_name="c", subcore_axis_name="s"),
    out_type=jax.ShapeDtypeStruct((NSER, 3, 16), jnp.float32),
    scratch_types=[
        pltpu.VMEM((M, 3 * W), jnp.float32),
        pltpu.VMEM((3, 16), jnp.float32),
        pltpu.VMEM((1, 16), jnp.int32),
    ],
)
def _sc_select(fused_hbm, out_hbm, buf, row, iscr):
    _sc_body(fused_hbm, out_hbm, buf, row, iscr)


def _recon_body(x_ref, sel_ref, season_ref, trend_ref):
    xb = x_ref[0]
    xT = jnp.transpose(xb)
    kf = sel_ref[:, 0, :]
    rf = sel_ref[:, 1, :]
    imf = sel_ref[:, 2, :]

    a_ar = lax.broadcasted_iota(jnp.int32, (C, U), 1)
    v_ar = lax.broadcasted_iota(jnp.int32, (C, V), 1)
    j_ar = lax.broadcasted_iota(jnp.int32, (C, 16), 1)
    season = jnp.zeros((C, U, V), jnp.float32)
    two_over_l = jnp.float32(2.0 / L)
    for j in range(TOP_K):
        jhit = j_ar == j
        ksel = jnp.sum(jnp.where(jhit, kf, 0.0), axis=1).astype(jnp.int32)
        re = jnp.sum(jnp.where(jhit, rf, 0.0), axis=1) * two_over_l
        im = jnp.sum(jnp.where(jhit, imf, 0.0), axis=1) * two_over_l
        ka = (ksel[:, None] * a_ar) & 63
        kv = (ksel[:, None] * v_ar) & 8191
        aa = ka.astype(jnp.float32) * jnp.float32(2.0 * np.pi / 64.0)
        bb = kv.astype(jnp.float32) * jnp.float32(2.0 * np.pi / 8192.0)
        ca, sa = jnp.cos(aa), jnp.sin(aa)
        cb, sb = jnp.cos(bb), jnp.sin(bb)
        p1 = (re[:, None] * ca - im[:, None] * sa)[:, :, None]
        p2 = -(re[:, None] * sa + im[:, None] * ca)[:, :, None]
        season = season + p1 * cb[:, None, :] + p2 * sb[:, None, :]

    season = season.reshape(C, L)
    season_ref[0] = season
    trend_ref[0] = xT - season


def _tc_recon(x, sel):
    return pl.pallas_call(
        _recon_body,
        grid=(NB,),
        in_specs=[
            pl.BlockSpec((1, L, C), lambda b: (b, 0, 0)),
            pl.BlockSpec((C, 3, 16), lambda b: (b, 0, 0)),
        ],
        out_specs=[
            pl.BlockSpec((1, C, L), lambda b: (b, 0, 0)),
            pl.BlockSpec((1, C, L), lambda b: (b, 0, 0)),
        ],
        out_shape=[
            jax.ShapeDtypeStruct((NB, C, L), jnp.float32),
            jax.ShapeDtypeStruct((NB, C, L), jnp.float32),
        ],
        compiler_params=pltpu.CompilerParams(
            dimension_semantics=("parallel",),
        ),
    )(x, sel)


@jax.jit
def kernel(x):
    fused = _tc_spectrum(x)
    sel = _sc_select(fused)
    season, trend = _tc_recon(x, sel)
    return (season, trend)


# speedup vs baseline: 4.3517x; 1.0002x over previous
import functools

import numpy as np
import jax
import jax.numpy as jnp
from jax import lax
from jax.experimental import pallas as pl
from jax.experimental.pallas import tpu as pltpu
from jax.experimental.pallas import tpu_sc as plsc

L = 8192
C = 64
NB = 32
V, U = 128, 64
M, W = 128, 16
TOP_K = 5
CUT = 1637
NSER = NB * C

_v = np.arange(V)
_u = np.arange(U)
_m = np.arange(M)
_w = np.arange(W)

_S1 = np.exp(-2j * np.pi * np.outer(_m, _v) / M)
_TW = np.exp(-2j * np.pi * np.outer(_m, _u) / L)
_S2 = np.exp(-2j * np.pi * np.outer(_u, _w) / U)

S1RE = np.ascontiguousarray(_S1.real, np.float32)
S1IM = np.ascontiguousarray(_S1.imag, np.float32)
TWRE = np.ascontiguousarray(_TW.real, np.float32)
TWIM = np.ascontiguousarray(_TW.imag, np.float32)
S2RE = np.ascontiguousarray(_S2.real, np.float32)
S2IM = np.ascontiguousarray(_S2.imag, np.float32)

_kval = (128 * _w[None, :] + _m[:, None]).astype(np.int32)
KVALID = np.ascontiguousarray((_kval >= 1) & (_kval <= CUT))


def _dft_body(x_ref, s1re, s1im, twre, twim, s2re, s2im, kvalid_ref, out_ref):
    xb = x_ref[0]
    xrp = jnp.transpose(xb.reshape(V, U, C), (0, 2, 1)).reshape(V, C * U)

    hp = jax.lax.Precision.HIGHEST
    are = lax.dot(s1re[...], xrp, precision=hp).reshape(M, C, U)
    aim = lax.dot(s1im[...], xrp, precision=hp).reshape(M, C, U)
    twr = twre[...].reshape(M, 1, U)
    twi = twim[...].reshape(M, 1, U)
    bre = are * twr - aim * twi
    bim = are * twi + aim * twr
    b2re = bre.reshape(M * C, U)
    b2im = bim.reshape(M * C, U)
    g2 = jnp.concatenate([s2re[...], s2im[...]], axis=1)
    p1 = lax.dot(b2re, g2, precision=hp)
    p2 = lax.dot(b2im, g2, precision=hp)
    cre = (p1[:, 0:W] - p2[:, W:2 * W]).reshape(M, C, W)
    cim = (p1[:, W:2 * W] + p2[:, 0:W]).reshape(M, C, W)

    mag2 = cre * cre + cim * cim
    kvalid = kvalid_ref[...].reshape(M, 1, W)
    mag2 = jnp.where(kvalid, mag2, -1.0)
    out_ref[0] = jnp.concatenate([mag2, cre, cim], axis=2)


def _tc_spectrum(x):
    return pl.pallas_call(
        _dft_body,
        grid=(NB,),
        in_specs=[
            pl.BlockSpec((1, L, C), lambda b: (b, 0, 0)),
            pl.BlockSpec((M, V), lambda b: (0, 0)),
            pl.BlockSpec((M, V), lambda b: (0, 0)),
            pl.BlockSpec((M, U), lambda b: (0, 0)),
            pl.BlockSpec((M, U), lambda b: (0, 0)),
            pl.BlockSpec((U, W), lambda b: (0, 0)),
            pl.BlockSpec((U, W), lambda b: (0, 0)),
            pl.BlockSpec((M, W), lambda b: (0, 0)),
        ],
        out_specs=pl.BlockSpec((1, M, C, 3 * W), lambda b: (b, 0, 0, 0)),
        out_shape=jax.ShapeDtypeStruct((NB, M, C, 3 * W), jnp.float32),
        compiler_params=pltpu.CompilerParams(
            dimension_semantics=("parallel",),
            vmem_limit_bytes=62 * 1024 * 1024,
        ),
    )(x, S1RE, S1IM, TWRE, TWIM, S2RE, S2IM, KVALID)


def _take16(x, idx):
    return x.at[idx].get(mode="promise_in_bounds")


def _sc_body(fused_hbm, out_hbm, buf, row):
    nc = 2
    wid = lax.axis_index("s") * nc + lax.axis_index("c")
    liota = lax.iota(jnp.int32, 16)

    def xlane_max(x):
        for sh in (8, 4, 2, 1):
            x = jnp.maximum(x, _take16(x, liota ^ sh))
        return x

    def xlane_min(x):
        for sh in (8, 4, 2, 1):
            x = jnp.minimum(x, _take16(x, liota ^ sh))
        return x

    def series_body(i, carry):
        pltpu.sync_copy(fused_hbm.at[wid, :, i, :], buf)

        kvec = jnp.zeros((16,), jnp.float32)
        revec = jnp.zeros((16,), jnp.float32)
        imvec = jnp.zeros((16,), jnp.float32)
        for j in range(TOP_K):
            def scan_body(m, mxam):
                mx, am = mxam
                vals = buf[m, 0:16]
                upd = vals > mx
                mx = jnp.where(upd, vals, mx)
                am = jnp.where(upd, jnp.full((16,), m, jnp.int32), am)
                return mx, am

            mx0 = jnp.full((16,), -2.0, jnp.float32)
            am0 = jnp.zeros((16,), jnp.int32)
            mx, am = lax.fori_loop(0, M, scan_body, (mx0, am0))
            gmax_v = xlane_max(mx)
            kcand = jnp.where(mx == gmax_v, 128 * liota + am, jnp.int32(4096))
            ks_v = xlane_min(kcand)
            ks = ks_v[0]
            m_star = ks & 127
            w_star_v = ks_v >> 7
            lane_hit = liota == w_star_v
            magrow = buf[m_star, 0:16]
            rerow = buf[m_star, 16:32]
            imrow = buf[m_star, 32:48]
            re_v = _take16(rerow, w_star_v)
            im_v = _take16(imrow, w_star_v)
            buf[m_star, 0:16] = jnp.where(lane_hit, -1.0, magrow)
            jsel = liota == j
            kvec = jnp.where(jsel, ks_v.astype(jnp.float32), kvec)
            revec = jnp.where(jsel, re_v, revec)
            imvec = jnp.where(jsel, im_v, imvec)
        row[0] = kvec
        row[1] = revec
        row[2] = imvec
        pltpu.sync_copy(row, out_hbm.at[wid * C + i])
        return carry

    lax.fori_loop(0, C, series_body, 0)


@functools.partial(
    pl.kernel,
    mesh=plsc.VectorSubcoreMesh(core_axis_name="c", subcore_axis_name="s"),
    out_type=jax.ShapeDtypeStruct((NSER, 3, 16), jnp.float32),
    scratch_types=[
        pltpu.VMEM((M, 3 * W), jnp.float32),
        pltpu.VMEM((3, 16), jnp.float32),
    ],
)
def _sc_select(fused_hbm, out_hbm, buf, row):
    _sc_body(fused_hbm, out_hbm, buf, row)


def _recon_body(x_ref, sel_ref, season_ref, trend_ref):
    xb = x_ref[0]
    xT = jnp.transpose(xb)
    kf = sel_ref[:, 0, :]
    rf = sel_ref[:, 1, :]
    imf = sel_ref[:, 2, :]

    a_ar = lax.broadcasted_iota(jnp.int32, (C, U), 1)
    v_ar = lax.broadcasted_iota(jnp.int32, (C, V), 1)
    j_ar = lax.broadcasted_iota(jnp.int32, (C, 16), 1)
    season = jnp.zeros((C, U, V), jnp.float32)
    two_over_l = jnp.float32(2.0 / L)
    for j in range(TOP_K):
        jhit = j_ar == j
        ksel = jnp.sum(jnp.where(jhit, kf, 0.0), axis=1).astype(jnp.int32)
        re = jnp.sum(jnp.where(jhit, rf, 0.0), axis=1) * two_over_l
        im = jnp.sum(jnp.where(jhit, imf, 0.0), axis=1) * two_over_l
        ka = (ksel[:, None] * a_ar) & 63
        kv = (ksel[:, None] * v_ar) & 8191
        aa = ka.astype(jnp.float32) * jnp.float32(2.0 * np.pi / 64.0)
        bb = kv.astype(jnp.float32) * jnp.float32(2.0 * np.pi / 8192.0)
        ca, sa = jnp.cos(aa), jnp.sin(aa)
        cb, sb = jnp.cos(bb), jnp.sin(bb)
        p1 = (re[:, None] * ca - im[:, None] * sa)[:, :, None]
        p2 = -(re[:, None] * sa + im[:, None] * ca)[:, :, None]
        season = season + p1 * cb[:, None, :] + p2 * sb[:, None, :]

    season = season.reshape(C, L)
    season_ref[0] = season
    trend_ref[0] = xT - season


def _tc_recon(x, sel):
    return pl.pallas_call(
        _recon_body,
        grid=(NB,),
        in_specs=[
            pl.BlockSpec((1, L, C), lambda b: (b, 0, 0)),
            pl.BlockSpec((C, 3, 16), lambda b: (b, 0, 0)),
        ],
        out_specs=[
            pl.BlockSpec((1, C, L), lambda b: (b, 0, 0)),
            pl.BlockSpec((1, C, L), lambda b: (b, 0, 0)),
        ],
        out_shape=[
            jax.ShapeDtypeStruct((NB, C, L), jnp.float32),
            jax.ShapeDtypeStruct((NB, C, L), jnp.float32),
        ],
        compiler_params=pltpu.CompilerParams(
            dimension_semantics=("parallel",),
        ),
    )(x, sel)


@jax.jit
def kernel(x):
    fused = _tc_spectrum(x)
    sel = _sc_select(fused)
    season, trend = _tc_recon(x, sel)
    return (season, trend)
